# baseline (speedup 1.0000x reference)
import jax
import jax.numpy as jnp
from jax import lax
from jax.experimental import pallas as pl
from jax.experimental.pallas import tpu as pltpu
from jax.experimental.pallas import tpu_sc as plsc

BATCH = 1024
VOCAB = 1000
EMB = 64
VPAD = 1024
NCB = VPAD // 128
LANES = 16
NC = 2
NS = 16
NW = NC * NS
BPW = BATCH // NW
NCH = VOCAB // LANES


def _tc_main_body(wc_ref, wo_ref, eg_ref):
    wc = jnp.concatenate(
        [wc_ref[...], jnp.zeros((VPAD - VOCAB, EMB), jnp.float32)], axis=0)
    wo = jnp.concatenate(
        [wo_ref[...], jnp.zeros((VPAD - VOCAB, EMB), jnp.float32)], axis=0)
    for cb in range(NCB):
        wo_cb = wo[cb * 128:(cb + 1) * 128, :]
        g_cb = lax.dot_general(wc, wo_cb, (((1,), (1,)), ((), ())),
                               preferred_element_type=jnp.float32)
        eg_ref[:, cb, :] = jnp.exp(g_cb)


_tc_main = pl.pallas_call(
    _tc_main_body,
    out_shape=jax.ShapeDtypeStruct((VPAD, NCB, 128), jnp.float32),
)


def _sc_lowsum_body(av_hbm, eg_hbm, co_hbm, lp_hbm, tp_hbm,
                    av_v, rows_v, co_v, lp_v, tp_v, sem):
    wid = lax.axis_index("s") * NC + lax.axis_index("c")
    base = wid * BPW
    d1 = pltpu.async_copy(av_hbm.at[pl.ds(base, BPW)], av_v, sem)
    d2 = pltpu.async_copy(co_hbm, co_v, sem)
    d1.wait()
    d2.wait()
    cv0 = co_v[pl.ds(base, LANES)]
    cv1 = co_v[pl.ds(base + LANES, LANES)]
    g1 = pltpu.async_copy(eg_hbm.at[cv0], rows_v.at[pl.ds(0, LANES)], sem)
    g2 = pltpu.async_copy(eg_hbm.at[cv1], rows_v.at[pl.ds(LANES, LANES)], sem)
    g1.wait()
    g2.wait()

    i16 = jnp.arange(LANES, dtype=jnp.int32)
    ov0 = co_v[pl.ds(BATCH + base, LANES)]
    ov1 = co_v[pl.ds(BATCH + base + LANES, LANES)]
    tp_v[0, pl.ds(0, LANES)] = plsc.load_gather(
        rows_v, [i16, lax.shift_right_logical(ov0, 7),
                 lax.bitwise_and(ov0, 127)])
    tp_v[0, pl.ds(LANES, LANES)] = plsc.load_gather(
        rows_v, [i16 + LANES, lax.shift_right_logical(ov1, 7),
                 lax.bitwise_and(ov1, 127)])
    pltpu.sync_copy(tp_v, tp_hbm.at[pl.ds(wid >> 2, 1),
                                    pl.ds((wid & 3) * 2 * LANES, 2 * LANES)])

    def row_body(rloc, _):
        rvec = jnp.full((LANES,), rloc, jnp.int32)
        partial = []
        for j in range(NCH):
            a = av_v[rloc, pl.ds(j * LANES, LANES)]
            partial.append(plsc.load_gather(
                rows_v, [rvec, lax.shift_right_logical(a, 7),
                         lax.bitwise_and(a, 127)]))
        a = av_v[rloc, pl.ds(VOCAB - LANES, LANES)]
        g = plsc.load_gather(
            rows_v, [rvec, lax.shift_right_logical(a, 7),
                     lax.bitwise_and(a, 127)])
        partial.append(jnp.where(i16 >= LANES - (VOCAB - NCH * LANES),
                                 g, 0.0))
        while len(partial) > 1:
            partial = [partial[i] + partial[i + 1]
                       for i in range(0, len(partial) - 1, 2)] \
                      + ([partial[-1]] if len(partial) % 2 else [])
        lp_v[rloc >> 3, pl.ds((rloc & 7) * LANES, LANES)] = partial[0]
        return 0

    lax.fori_loop(0, BPW, row_body, 0)
    pltpu.sync_copy(lp_v, lp_hbm.at[pl.ds(wid * (BPW * LANES // 128),
                                          BPW * LANES // 128)])


_sc_lowsum = pl.kernel(
    _sc_lowsum_body,
    out_type=[jax.ShapeDtypeStruct((BATCH * LANES // 128, 128), jnp.float32),
              jax.ShapeDtypeStruct((BATCH // 128, 128), jnp.float32)],
    mesh=plsc.VectorSubcoreMesh(core_axis_name="c", subcore_axis_name="s"),
    scratch_types=[pltpu.VMEM((BPW, VOCAB), jnp.int32),
                   pltpu.VMEM((BPW, NCB, 128), jnp.float32),
                   pltpu.VMEM((2 * BATCH,), jnp.int32),
                   pltpu.VMEM((BPW * LANES // 128, 128), jnp.float32),
                   pltpu.VMEM((1, 2 * LANES), jnp.float32),
                   pltpu.SemaphoreType.DMA],
    compiler_params=pltpu.CompilerParams(use_tc_tiling_on_sc=False,
                                         needs_layout_passes=False),
)


def _tc_final_body(lp_ref, top_ref, out_ref):
    cc = lax.broadcasted_iota(jnp.int32, (128, 128 // LANES), 0)
    gg = lax.broadcasted_iota(jnp.int32, (128, 128 // LANES), 1)
    sel = (lax.shift_right_logical(cc, 4) == gg).astype(jnp.float32)
    low = jnp.dot(lp_ref[...], sel,
                  preferred_element_type=jnp.float32)
    total = jnp.sum(jnp.log(low)) - jnp.sum(jnp.log(top_ref[...]))
    out_ref[...] = (total / BATCH).reshape(1, 1)


_tc_final = pl.pallas_call(
    _tc_final_body,
    out_shape=jax.ShapeDtypeStruct((1, 1), jnp.float32),
)


def kernel(center, outside, all_vocabs, W_center, W_outside):
    co = jnp.concatenate([center.reshape(BATCH), outside.reshape(BATCH)])
    eg = _tc_main(W_center, W_outside)
    lp, tp = _sc_lowsum(all_vocabs, eg, co)
    loss = _tc_final(lp, tp)
    return loss[0, 0]


# speedup vs baseline: 1.0660x; 1.0660x over previous
import jax
import jax.numpy as jnp
from jax import lax
from jax.experimental import pallas as pl
from jax.experimental.pallas import tpu as pltpu
from jax.experimental.pallas import tpu_sc as plsc

BATCH = 1024
VOCAB = 1000
EMB = 64
VPAD = 1024
NCB = VPAD // 128
LANES = 16
NC = 2
NS = 16
NW = NC * NS
BPW = BATCH // NW
HALF = BPW // 2


def _tc_main_body(wc_ref, wo_ref, avt_ref, eg_ref, av3_ref):
    wc = jnp.concatenate(
        [wc_ref[...], jnp.zeros((VPAD - VOCAB, EMB), jnp.float32)], axis=0)
    wo = jnp.concatenate(
        [wo_ref[...], jnp.zeros((VPAD - VOCAB, EMB), jnp.float32)], axis=0)
    ntail = VOCAB - (NCB - 1) * 128
    for cb in range(NCB):
        wo_cb = wo[cb * 128:(cb + 1) * 128, :]
        g_cb = lax.dot_general(wc, wo_cb, (((1,), (1,)), ((), ())),
                               preferred_element_type=jnp.float32)
        e_cb = jnp.exp(g_cb)
        if cb == NCB - 1:
            ccol = lax.broadcasted_iota(jnp.int32, (VPAD, 128), 1)
            e_cb = jnp.where(ccol < ntail, e_cb, 0.0)
        eg_ref[:, cb, :] = e_cb
        if cb == NCB - 1:
            blk = lax.transpose(avt_ref[cb * 128:VOCAB, :], (1, 0))
            av3_ref[cb, :, :] = jnp.concatenate(
                [blk, jnp.full((BATCH, 128 - ntail), VOCAB, jnp.int32)],
                axis=1)
        else:
            av3_ref[cb, :, :] = lax.transpose(
                avt_ref[cb * 128:(cb + 1) * 128, :], (1, 0))


_tc_main = pl.pallas_call(
    _tc_main_body,
    out_shape=[jax.ShapeDtypeStruct((VPAD, NCB, 128), jnp.float32),
               jax.ShapeDtypeStruct((NCB, BATCH, 128), jnp.int32)],
)


def _sc_lowsum_body(cidx_hbm, oidx_hbm, av_hbm, eg_hbm, lp_hbm, tp_hbm,
                    cidx_v, oidx_v, av_v, rows_v, lp_v, tp_v,
                    sem_a, sem_g1, sem_g2):
    wid = lax.axis_index("s") * NC + lax.axis_index("c")
    base = wid * BPW
    d_c = pltpu.async_copy(cidx_hbm.at[pl.ds(base, BPW)], cidx_v, sem_a)
    d_o = pltpu.async_copy(oidx_hbm.at[pl.ds(base, BPW)], oidx_v, sem_a)
    av_copies = [pltpu.async_copy(av_hbm.at[cb, pl.ds(base, BPW)],
                                  av_v.at[cb], sem_a)
                 for cb in range(NCB)]
    d_c.wait()
    d_o.wait()
    i16 = jnp.arange(LANES, dtype=jnp.int32)
    z16 = jnp.zeros((LANES,), jnp.int32)
    cv0 = plsc.load_gather(cidx_v, [i16, z16])
    cv1 = plsc.load_gather(cidx_v, [i16 + LANES, z16])
    g1 = pltpu.async_copy(eg_hbm.at[cv0], rows_v.at[pl.ds(0, HALF)], sem_g1)
    g2 = pltpu.async_copy(eg_hbm.at[cv1], rows_v.at[pl.ds(HALF, HALF)],
                          sem_g2)
    for c in av_copies:
        c.wait()

    def row_body(rloc, _):
        rvec = jnp.full((LANES,), rloc, jnp.int32)
        partial = []
        for cb in range(NCB):
            for j in range(128 // LANES):
                a = av_v[cb, rloc, pl.ds(j * LANES, LANES)]
                partial.append(plsc.load_gather(
                    rows_v, [rvec, lax.shift_right_logical(a, 7),
                             lax.bitwise_and(a, 127)]))
        while len(partial) > 1:
            partial = [partial[i] + partial[i + 1]
                       for i in range(0, len(partial) - 1, 2)] \
                      + ([partial[-1]] if len(partial) % 2 else [])
        lp_v[rloc >> 3, pl.ds((rloc & 7) * LANES, LANES)] = partial[0]
        return 0

    g1.wait()
    ov0 = plsc.load_gather(oidx_v, [i16, z16])
    tp_v[0, pl.ds(0, LANES)] = plsc.load_gather(
        rows_v, [i16, lax.shift_right_logical(ov0, 7),
                 lax.bitwise_and(ov0, 127)])
    lax.fori_loop(0, HALF, row_body, 0)
    g2.wait()
    ov1 = plsc.load_gather(oidx_v, [i16 + LANES, z16])
    tp_v[0, pl.ds(LANES, LANES)] = plsc.load_gather(
        rows_v, [i16 + LANES, lax.shift_right_logical(ov1, 7),
                 lax.bitwise_and(ov1, 127)])
    lax.fori_loop(HALF, BPW, row_body, 0)

    pltpu.sync_copy(tp_v, tp_hbm.at[pl.ds(wid >> 2, 1),
                                    pl.ds((wid & 3) * 2 * LANES, 2 * LANES)])
    pltpu.sync_copy(lp_v, lp_hbm.at[pl.ds(wid * (BPW * LANES // 128),
                                          BPW * LANES // 128)])


_sc_lowsum = pl.kernel(
    _sc_lowsum_body,
    out_type=[jax.ShapeDtypeStruct((BATCH * LANES // 128, 128), jnp.float32),
              jax.ShapeDtypeStruct((BATCH // 128, 128), jnp.float32)],
    mesh=plsc.VectorSubcoreMesh(core_axis_name="c", subcore_axis_name="s"),
    scratch_types=[pltpu.VMEM((BPW, 1), jnp.int32),
                   pltpu.VMEM((BPW, 1), jnp.int32),
                   pltpu.VMEM((NCB, BPW, 128), jnp.int32),
                   pltpu.VMEM((BPW, NCB, 128), jnp.float32),
                   pltpu.VMEM((BPW * LANES // 128, 128), jnp.float32),
                   pltpu.VMEM((1, 2 * LANES), jnp.float32),
                   pltpu.SemaphoreType.DMA,
                   pltpu.SemaphoreType.DMA,
                   pltpu.SemaphoreType.DMA],
    compiler_params=pltpu.CompilerParams(use_tc_tiling_on_sc=False,
                                         needs_layout_passes=False),
)


def _tc_final_body(lp_ref, top_ref, out_ref):
    cc = lax.broadcasted_iota(jnp.int32, (128, 128 // LANES), 0)
    gg = lax.broadcasted_iota(jnp.int32, (128, 128 // LANES), 1)
    sel = (lax.shift_right_logical(cc, 4) == gg).astype(jnp.float32)
    low = jnp.dot(lp_ref[...], sel,
                  preferred_element_type=jnp.float32)
    total = jnp.sum(jnp.log(low)) - jnp.sum(jnp.log(top_ref[...]))
    out_ref[...] = (total / BATCH).reshape(1, 1)


_tc_final = pl.pallas_call(
    _tc_final_body,
    out_shape=jax.ShapeDtypeStruct((1, 1), jnp.float32),
)


def kernel(center, outside, all_vocabs, W_center, W_outside):
    eg, av3 = _tc_main(W_center, W_outside, all_vocabs.T)
    lp, tp = _sc_lowsum(center, outside, av3, eg)
    loss = _tc_final(lp, tp)
    return loss[0, 0]


# speedup vs baseline: 1.1673x; 1.0950x over previous
import jax
import jax.numpy as jnp
from jax import lax
from jax.experimental import pallas as pl
from jax.experimental.pallas import tpu as pltpu
from jax.experimental.pallas import tpu_sc as plsc

BATCH = 1024
VOCAB = 1000
EMB = 64
VPAD = 1024
NCB = VPAD // 128
LANES = 16
NC = 2
NS = 16
NW = NC * NS
BPW = BATCH // NW
HALF = BPW // 2


def _tc_main_body(wc_ref, wo_ref, avt_ref, eg_ref, av3_ref):
    wc = jnp.concatenate(
        [wc_ref[...], jnp.zeros((VPAD - VOCAB, EMB), jnp.float32)], axis=0)
    wo = jnp.concatenate(
        [wo_ref[...], jnp.zeros((VPAD - VOCAB, EMB), jnp.float32)], axis=0)
    ntail = VOCAB - (NCB - 1) * 128
    for cb in range(NCB):
        wo_cb = wo[cb * 128:(cb + 1) * 128, :]
        g_cb = lax.dot_general(wc, wo_cb, (((1,), (1,)), ((), ())),
                               preferred_element_type=jnp.float32)
        e_cb = jnp.exp(g_cb)
        if cb == NCB - 1:
            ccol = lax.broadcasted_iota(jnp.int32, (VPAD, 128), 1)
            e_cb = jnp.where(ccol < ntail, e_cb, 0.0)
        eg_ref[:, cb, :] = e_cb
        if cb == NCB - 1:
            blk = lax.transpose(avt_ref[cb * 128:VOCAB, :], (1, 0))
            av3_ref[cb, :, :] = jnp.concatenate(
                [blk, jnp.full((BATCH, 128 - ntail), VOCAB, jnp.int32)],
                axis=1)
        else:
            av3_ref[cb, :, :] = lax.transpose(
                avt_ref[cb * 128:(cb + 1) * 128, :], (1, 0))


_tc_main = pl.pallas_call(
    _tc_main_body,
    out_shape=[jax.ShapeDtypeStruct((VPAD, NCB, 128), jnp.float32),
               jax.ShapeDtypeStruct((NCB, BATCH, 128), jnp.int32)],
)


def _sc_lowsum_body(cidx_hbm, oidx_hbm, av_hbm, eg_hbm, lp_hbm, tp_hbm,
                    cidx_v, oidx_v, av_v, rows_v, lp_v, tp_v,
                    sem_a, sem_g1, sem_g2):
    wid = lax.axis_index("s") * NC + lax.axis_index("c")
    base = wid * BPW
    d_c = pltpu.async_copy(cidx_hbm.at[0, pl.ds(base, BPW)], cidx_v, sem_a)
    d_o = pltpu.async_copy(oidx_hbm.at[0, pl.ds(base, BPW)], oidx_v, sem_a)
    av_copies = [pltpu.async_copy(av_hbm.at[cb, pl.ds(base, BPW)],
                                  av_v.at[cb], sem_a)
                 for cb in range(NCB)]
    d_c.wait()
    d_o.wait()
    i16 = jnp.arange(LANES, dtype=jnp.int32)
    cv0 = cidx_v[pl.ds(0, LANES)]
    cv1 = cidx_v[pl.ds(LANES, LANES)]
    g1 = pltpu.async_copy(eg_hbm.at[cv0], rows_v.at[pl.ds(0, HALF)], sem_g1)
    g2 = pltpu.async_copy(eg_hbm.at[cv1], rows_v.at[pl.ds(HALF, HALF)],
                          sem_g2)
    for c in av_copies:
        c.wait()

    def row_body(rloc, _):
        rvec = jnp.full((LANES,), rloc, jnp.int32)
        partial = []
        for cb in range(NCB):
            for j in range(128 // LANES):
                a = av_v[cb, rloc, pl.ds(j * LANES, LANES)]
                partial.append(plsc.load_gather(
                    rows_v, [rvec, lax.shift_right_logical(a, 7),
                             lax.bitwise_and(a, 127)]))
        while len(partial) > 1:
            partial = [partial[i] + partial[i + 1]
                       for i in range(0, len(partial) - 1, 2)] \
                      + ([partial[-1]] if len(partial) % 2 else [])
        lp_v[rloc >> 3, pl.ds((rloc & 7) * LANES, LANES)] = partial[0]
        return 0

    g1.wait()
    ov0 = oidx_v[pl.ds(0, LANES)]
    tp_v[0, pl.ds(0, LANES)] = plsc.load_gather(
        rows_v, [i16, lax.shift_right_logical(ov0, 7),
                 lax.bitwise_and(ov0, 127)])
    lax.fori_loop(0, HALF, row_body, 0)
    g2.wait()
    ov1 = oidx_v[pl.ds(LANES, LANES)]
    tp_v[0, pl.ds(LANES, LANES)] = plsc.load_gather(
        rows_v, [i16 + LANES, lax.shift_right_logical(ov1, 7),
                 lax.bitwise_and(ov1, 127)])
    lax.fori_loop(HALF, BPW, row_body, 0)

    pltpu.sync_copy(tp_v, tp_hbm.at[pl.ds(wid >> 2, 1),
                                    pl.ds((wid & 3) * 2 * LANES, 2 * LANES)])
    pltpu.sync_copy(lp_v, lp_hbm.at[pl.ds(wid * (BPW * LANES // 128),
                                          BPW * LANES // 128)])


_sc_lowsum = pl.kernel(
    _sc_lowsum_body,
    out_type=[jax.ShapeDtypeStruct((BATCH * LANES // 128, 128), jnp.float32),
              jax.ShapeDtypeStruct((BATCH // 128, 128), jnp.float32)],
    mesh=plsc.VectorSubcoreMesh(core_axis_name="c", subcore_axis_name="s"),
    scratch_types=[pltpu.VMEM((BPW,), jnp.int32),
                   pltpu.VMEM((BPW,), jnp.int32),
                   pltpu.VMEM((NCB, BPW, 128), jnp.int32),
                   pltpu.VMEM((BPW, NCB, 128), jnp.float32),
                   pltpu.VMEM((BPW * LANES // 128, 128), jnp.float32),
                   pltpu.VMEM((1, 2 * LANES), jnp.float32),
                   pltpu.SemaphoreType.DMA,
                   pltpu.SemaphoreType.DMA,
                   pltpu.SemaphoreType.DMA],
    compiler_params=pltpu.CompilerParams(use_tc_tiling_on_sc=False,
                                         needs_layout_passes=False),
)


def _tc_final_body(lp_ref, top_ref, out_ref):
    cc = lax.broadcasted_iota(jnp.int32, (128, 128 // LANES), 0)
    gg = lax.broadcasted_iota(jnp.int32, (128, 128 // LANES), 1)
    sel = (lax.shift_right_logical(cc, 4) == gg).astype(jnp.float32)
    low = jnp.dot(lp_ref[...], sel,
                  preferred_element_type=jnp.float32)
    total = jnp.sum(jnp.log(low)) - jnp.sum(jnp.log(top_ref[...]))
    out_ref[...] = (total / BATCH).reshape(1, 1)


_tc_final = pl.pallas_call(
    _tc_final_body,
    out_shape=jax.ShapeDtypeStruct((1, 1), jnp.float32),
)


def kernel(center, outside, all_vocabs, W_center, W_outside):
    eg, av3 = _tc_main(W_center, W_outside, all_vocabs.T)
    lp, tp = _sc_lowsum(center.T, outside.T, av3, eg)
    loss = _tc_final(lp, tp)
    return loss[0, 0]
